# baseline (speedup 1.0000x reference)
import jax
import jax.numpy as jnp
from jax.experimental import pallas as pl
from jax.experimental.pallas import tpu as pltpu

_GROUPS_PER_BLOCK = 256


def _matvec_body(x_ref, p_ref, o_ref):
    y = jax.lax.dot_general(
        p_ref[...], x_ref[...],
        dimension_numbers=(((2,), (2,)), ((0,), (0,))),
        preferred_element_type=jnp.float32,
    )
    o_ref[...] = y.reshape(o_ref.shape)


def kernel(x, projection):
    b, n, d = x.shape
    rows = b * n
    groups = rows // d
    xf = x.reshape(groups, d, d)
    pb = jnp.broadcast_to(projection.reshape(1, 1, d), (groups, 1, d))
    grid = groups // _GROUPS_PER_BLOCK
    out = pl.pallas_call(
        _matvec_body,
        grid=(grid,),
        in_specs=[
            pl.BlockSpec((_GROUPS_PER_BLOCK, d, d), lambda i: (i, 0, 0)),
            pl.BlockSpec((_GROUPS_PER_BLOCK, 1, d), lambda i: (i, 0, 0)),
        ],
        out_specs=pl.BlockSpec((_GROUPS_PER_BLOCK, d), lambda i: (i, 0)),
        out_shape=jax.ShapeDtypeStruct((groups, d), jnp.float32),
    )(xf, pb)
    return out.reshape(b, n, 1)


# speedup vs baseline: 1.0679x; 1.0679x over previous
import functools

import jax
import jax.numpy as jnp
from jax import lax
from jax.experimental import pallas as pl
from jax.experimental.pallas import tpu as pltpu

_G = 128
_NBUF = 4
_D = 128


def _body(x_hbm, p_ref, o_hbm, xbuf, obuf, insem, outsem, *, nstep):
    def in_copy(step, slot):
        return pltpu.make_async_copy(
            x_hbm.at[pl.ds(step * _G, _G)], xbuf.at[slot], insem.at[slot])

    def out_copy(step, slot):
        return pltpu.make_async_copy(
            obuf.at[slot], o_hbm.at[pl.ds(step * _G, _G)], outsem.at[slot])

    for s in range(_NBUF):
        in_copy(s, s).start()

    def outer(i, _):
        for b in range(_NBUF):
            step = i * _NBUF + b
            in_copy(step, b).wait()

            @pl.when(step >= _NBUF)
            def _():
                out_copy(step - _NBUF, b).wait()

            y = lax.dot_general(
                p_ref[...], xbuf[b],
                dimension_numbers=(((2,), (2,)), ((0,), (0,))),
                preferred_element_type=jnp.float32,
            )
            obuf[b] = y.reshape(_G, _D)
            out_copy(step, b).start()

            @pl.when(step + _NBUF < nstep)
            def _():
                in_copy(step + _NBUF, b).start()
        return 0

    lax.fori_loop(0, nstep // _NBUF, outer, 0)
    for b in range(_NBUF):
        out_copy(nstep - _NBUF + b, b).wait()


def kernel(x, projection):
    b, n, d = x.shape
    rows = b * n
    groups = rows // d
    nstep = groups // _G
    xf = x.reshape(groups, d, d)
    pb = jnp.broadcast_to(projection.reshape(1, 1, d), (_G, 1, d))
    out = pl.pallas_call(
        functools.partial(_body, nstep=nstep),
        in_specs=[
            pl.BlockSpec(memory_space=pl.ANY),
            pl.BlockSpec(memory_space=pltpu.VMEM),
        ],
        out_specs=pl.BlockSpec(memory_space=pl.ANY),
        out_shape=jax.ShapeDtypeStruct((groups, d), jnp.float32),
        scratch_shapes=[
            pltpu.VMEM((_NBUF, _G, d, d), jnp.float32),
            pltpu.VMEM((_NBUF, _G, d), jnp.float32),
            pltpu.SemaphoreType.DMA((_NBUF,)),
            pltpu.SemaphoreType.DMA((_NBUF,)),
        ],
    )(xf, pb)
    return out.reshape(b, n, 1)


# speedup vs baseline: 1.0766x; 1.0081x over previous
import functools

import jax
import jax.numpy as jnp
from jax import lax
from jax.experimental import pallas as pl
from jax.experimental.pallas import tpu as pltpu

_G = 64
_NBUF = 8
_D = 128


def _body(x_hbm, p_ref, o_hbm, xbuf, obuf, insem, outsem, *, nstep):
    def in_copy(step, slot):
        return pltpu.make_async_copy(
            x_hbm.at[pl.ds(step * _G, _G)], xbuf.at[slot], insem.at[slot])

    def out_copy(step, slot):
        return pltpu.make_async_copy(
            obuf.at[slot], o_hbm.at[pl.ds(step * _G, _G)], outsem.at[slot])

    for s in range(_NBUF):
        in_copy(s, s).start()

    def outer(i, _):
        for b in range(_NBUF):
            step = i * _NBUF + b
            in_copy(step, b).wait()

            @pl.when(step >= _NBUF)
            def _():
                out_copy(step - _NBUF, b).wait()

            y = lax.dot_general(
                p_ref[...], xbuf[b],
                dimension_numbers=(((2,), (2,)), ((0,), (0,))),
                preferred_element_type=jnp.float32,
            )
            obuf[b] = y.reshape(_G, _D)
            out_copy(step, b).start()

            @pl.when(step + _NBUF < nstep)
            def _():
                in_copy(step + _NBUF, b).start()
        return 0

    lax.fori_loop(0, nstep // _NBUF, outer, 0)
    for b in range(_NBUF):
        out_copy(nstep - _NBUF + b, b).wait()


def kernel(x, projection):
    b, n, d = x.shape
    rows = b * n
    groups = rows // d
    nstep = groups // _G
    xf = x.reshape(groups, d, d)
    pb = jnp.broadcast_to(projection.reshape(1, 1, d), (_G, 1, d))
    out = pl.pallas_call(
        functools.partial(_body, nstep=nstep),
        in_specs=[
            pl.BlockSpec(memory_space=pl.ANY),
            pl.BlockSpec(memory_space=pltpu.VMEM),
        ],
        out_specs=pl.BlockSpec(memory_space=pl.ANY),
        out_shape=jax.ShapeDtypeStruct((groups, d), jnp.float32),
        scratch_shapes=[
            pltpu.VMEM((_NBUF, _G, d, d), jnp.float32),
            pltpu.VMEM((_NBUF, _G, d), jnp.float32),
            pltpu.SemaphoreType.DMA((_NBUF,)),
            pltpu.SemaphoreType.DMA((_NBUF,)),
        ],
    )(xf, pb)
    return out.reshape(b, n, 1)
